# baseline (speedup 1.0000x reference)
import functools

import jax
import jax.numpy as jnp
from jax import lax
from jax.experimental import pallas as pl
from jax.experimental.pallas import tpu as pltpu
from jax.experimental.pallas import tpu_sc as plsc

P, D, DMLP, E = 2048, 768, 3072, 8
TM = 1536
NT = DMLP // TM
NW = 32
PPW = P // NW
LANES = 16
NCHUNK = PPW // LANES


def _gate_body(x_ref, wg_ref, gt_ref):
    gt_ref[...] = lax.dot_general(
        wg_ref[...], x_ref[...], (((0,), (1,)), ((), ())),
        preferred_element_type=jnp.float32)


def _router_body(gt_hbm, out_hbm, gbuf, accv):
    wid = lax.axis_index("s") * 2 + lax.axis_index("c")
    base = wid * PPW
    for i in range(E):
        pltpu.sync_copy(gt_hbm.at[i, pl.ds(base, PPW)], gbuf.at[i])
    zero = jnp.zeros((LANES,), jnp.float32)
    acc = [zero for _ in range(E)]
    for c in range(NCHUNK):
        sl = pl.ds(c * LANES, LANES)
        vs = [gbuf[i, sl] for i in range(E)]
        m1 = vs[0]
        i1 = jnp.zeros((LANES,), jnp.int32)
        for i in range(1, E):
            gt = vs[i] > m1
            i1 = jnp.where(gt, i, i1)
            m1 = jnp.where(gt, vs[i], m1)
        m2 = jnp.full((LANES,), -jnp.inf, jnp.float32)
        i2 = jnp.zeros((LANES,), jnp.int32)
        for i in range(E):
            gt = (vs[i] > m2) & (i1 != i)
            i2 = jnp.where(gt, i, i2)
            m2 = jnp.where(gt, vs[i], m2)
        r = jnp.exp(m2 - m1)
        w1 = 1.0 / (1.0 + r)
        w2 = r / (1.0 + r)
        for i in range(E):
            acc[i] = (acc[i] + jnp.where(i1 == i, w1, 0.0)
                      + jnp.where(i2 == i, w2, 0.0))
    for i in range(E):
        accv[i, pl.ds(0, LANES)] = acc[i]
    pltpu.sync_copy(accv, out_hbm.at[wid])


def _moe_body(x_ref, coef_ref, win_ref, bin_ref, wout_ref, bout_ref,
              out_ref, cbuf_ref):
    e = pl.program_id(0)
    t = pl.program_id(1)

    @pl.when((e == 0) & (t == 0))
    def _init():
        row = jnp.sum(coef_ref[...], axis=0, keepdims=True)
        grp = jax.lax.broadcasted_iota(jnp.int32, (1, E * LANES), 1) // LANES
        cs = []
        for i in range(E):
            ci = jnp.sum(jnp.where(grp == i, row, 0.0),
                         axis=1, keepdims=True)
            cs.append(ci)
            cbuf_ref[i:i + 1, :] = jnp.broadcast_to(ci, (1, 128))
        bias = cs[0] * bout_ref[0:1, :]
        for i in range(1, E):
            bias = bias + cs[i] * bout_ref[i:i + 1, :]
        out_ref[...] = jnp.broadcast_to(bias, out_ref.shape)

    c11 = cbuf_ref[pl.ds(e, 1), 0:1]
    pre = jnp.dot(x_ref[...], win_ref[0],
                  preferred_element_type=jnp.float32) + bin_ref[0]
    h = jnp.maximum(pre, 0.0)
    out_ref[...] += jnp.dot(h, wout_ref[0] * c11,
                            preferred_element_type=jnp.float32)


@jax.jit
def kernel(x, W_gate, W_in, b_in, W_out, b_out):
    B = x.shape[0]
    x2 = x.reshape(B * P, D)
    b_in3 = b_in.reshape(E, 1, DMLP)

    gt = pl.pallas_call(
        _gate_body,
        in_specs=[pl.BlockSpec((B * P, D), lambda: (0, 0)),
                  pl.BlockSpec((D, E), lambda: (0, 0))],
        out_specs=pl.BlockSpec((E, B * P), lambda: (0, 0)),
        out_shape=jax.ShapeDtypeStruct((E, B * P), jnp.float32),
        grid=(),
    )(x2, W_gate)

    router = pl.kernel(
        _router_body,
        mesh=plsc.VectorSubcoreMesh(core_axis_name="c", subcore_axis_name="s"),
        out_type=jax.ShapeDtypeStruct((NW, E, LANES), jnp.float32),
        scratch_types=[
            pltpu.VMEM((E, PPW), jnp.float32),
            pltpu.VMEM((E, LANES), jnp.float32),
        ],
    )
    partial_coefs = router(gt).reshape(NW, E * LANES)

    out = pl.pallas_call(
        _moe_body,
        grid=(E, NT),
        in_specs=[
            pl.BlockSpec((B * P, D), lambda e, t: (0, 0)),
            pl.BlockSpec((NW, E * LANES), lambda e, t: (0, 0)),
            pl.BlockSpec((1, D, TM), lambda e, t: (e, 0, t)),
            pl.BlockSpec((1, 1, TM), lambda e, t: (e, 0, t)),
            pl.BlockSpec((1, TM, D), lambda e, t: (e, t, 0)),
            pl.BlockSpec((E, D), lambda e, t: (0, 0)),
        ],
        out_specs=pl.BlockSpec((B * P, D), lambda e, t: (0, 0)),
        out_shape=jax.ShapeDtypeStruct((B * P, D), jnp.float32),
        scratch_shapes=[pltpu.VMEM((E, 128), jnp.float32)],
        compiler_params=pltpu.CompilerParams(
            dimension_semantics=("arbitrary", "arbitrary")),
    )(x2, partial_coefs, W_in, b_in3, W_out, b_out)
    return out.reshape(B, P, D)


# speedup vs baseline: 1.0132x; 1.0132x over previous
import functools

import jax
import jax.numpy as jnp
from jax import lax
from jax.experimental import pallas as pl
from jax.experimental.pallas import tpu as pltpu
from jax.experimental.pallas import tpu_sc as plsc

P, D, DMLP, E = 2048, 768, 3072, 8
TM = 1536
NT = DMLP // TM
NW = 32
PPW = P // NW
LANES = 16
NCHUNK = PPW // LANES


def _gate_body(x_ref, wg_ref, gt_ref):
    gt_ref[...] = lax.dot_general(
        wg_ref[...], x_ref[...], (((0,), (1,)), ((), ())),
        preferred_element_type=jnp.float32)


def _router_body(gt_hbm, out_hbm, gbuf, accv, sem):
    wid = lax.axis_index("s") * 2 + lax.axis_index("c")
    base = wid * PPW
    hs = [pltpu.async_copy(gt_hbm.at[i, pl.ds(base, PPW)], gbuf.at[i], sem)
          for i in range(E)]
    for h in hs:
        h.wait()
    zero = jnp.zeros((LANES,), jnp.float32)
    acc = [zero for _ in range(E)]
    for c in range(NCHUNK):
        sl = pl.ds(c * LANES, LANES)
        vs = [gbuf[i, sl] for i in range(E)]
        m1 = vs[0]
        i1 = jnp.zeros((LANES,), jnp.int32)
        for i in range(1, E):
            gt = vs[i] > m1
            i1 = jnp.where(gt, i, i1)
            m1 = jnp.where(gt, vs[i], m1)
        m2 = jnp.full((LANES,), -jnp.inf, jnp.float32)
        i2 = jnp.zeros((LANES,), jnp.int32)
        for i in range(E):
            gt = (vs[i] > m2) & (i1 != i)
            i2 = jnp.where(gt, i, i2)
            m2 = jnp.where(gt, vs[i], m2)
        r = jnp.exp(m2 - m1)
        w1 = 1.0 / (1.0 + r)
        w2 = r / (1.0 + r)
        for i in range(E):
            acc[i] = (acc[i] + jnp.where(i1 == i, w1, 0.0)
                      + jnp.where(i2 == i, w2, 0.0))
    for i in range(E):
        accv[i, pl.ds(0, LANES)] = acc[i]
    pltpu.sync_copy(accv, out_hbm.at[wid])


def _moe_body(x_ref, coef_ref, win_ref, bin_ref, wout_ref, bout_ref,
              out_ref, cbuf_ref):
    e = pl.program_id(0)
    t = pl.program_id(1)

    @pl.when((e == 0) & (t == 0))
    def _init():
        row = jnp.sum(coef_ref[...], axis=0, keepdims=True)
        grp = jax.lax.broadcasted_iota(jnp.int32, (1, E * LANES), 1) // LANES
        cs = []
        for i in range(E):
            ci = jnp.sum(jnp.where(grp == i, row, 0.0),
                         axis=1, keepdims=True)
            cs.append(ci)
            cbuf_ref[i:i + 1, :] = jnp.broadcast_to(ci, (1, 128))
        bias = cs[0] * bout_ref[0:1, :]
        for i in range(1, E):
            bias = bias + cs[i] * bout_ref[i:i + 1, :]
        out_ref[...] = jnp.broadcast_to(bias, out_ref.shape)

    c11 = cbuf_ref[pl.ds(e, 1), 0:1]
    pre = jnp.dot(x_ref[...], win_ref[0],
                  preferred_element_type=jnp.float32) + bin_ref[0]
    h = jnp.maximum(pre, 0.0)
    out_ref[...] += jnp.dot(h, wout_ref[0] * c11,
                            preferred_element_type=jnp.float32)


@jax.jit
def kernel(x, W_gate, W_in, b_in, W_out, b_out):
    B = x.shape[0]
    x2 = x.reshape(B * P, D)
    b_in3 = b_in.reshape(E, 1, DMLP)

    gt = pl.pallas_call(
        _gate_body,
        in_specs=[pl.BlockSpec((B * P, D), lambda: (0, 0)),
                  pl.BlockSpec((D, E), lambda: (0, 0))],
        out_specs=pl.BlockSpec((E, B * P), lambda: (0, 0)),
        out_shape=jax.ShapeDtypeStruct((E, B * P), jnp.float32),
        grid=(),
    )(x2, W_gate)

    router = pl.kernel(
        _router_body,
        mesh=plsc.VectorSubcoreMesh(core_axis_name="c", subcore_axis_name="s"),
        out_type=jax.ShapeDtypeStruct((NW, E, LANES), jnp.float32),
        scratch_types=[
            pltpu.VMEM((E, PPW), jnp.float32),
            pltpu.VMEM((E, LANES), jnp.float32),
            pltpu.SemaphoreType.DMA,
        ],
    )
    partial_coefs = router(gt).reshape(NW, E * LANES)

    out = pl.pallas_call(
        _moe_body,
        grid=(E, NT),
        in_specs=[
            pl.BlockSpec((B * P, D), lambda e, t: (0, 0)),
            pl.BlockSpec((NW, E * LANES), lambda e, t: (0, 0)),
            pl.BlockSpec((1, D, TM), lambda e, t: (e, 0, t)),
            pl.BlockSpec((1, 1, TM), lambda e, t: (e, 0, t)),
            pl.BlockSpec((1, TM, D), lambda e, t: (e, t, 0)),
            pl.BlockSpec((E, D), lambda e, t: (0, 0)),
        ],
        out_specs=pl.BlockSpec((B * P, D), lambda e, t: (0, 0)),
        out_shape=jax.ShapeDtypeStruct((B * P, D), jnp.float32),
        scratch_shapes=[pltpu.VMEM((E, 128), jnp.float32)],
        compiler_params=pltpu.CompilerParams(
            dimension_semantics=("arbitrary", "arbitrary")),
    )(x2, partial_coefs, W_in, b_in3, W_out, b_out)
    return out.reshape(B, P, D)


# speedup vs baseline: 1.0136x; 1.0004x over previous
import jax
import jax.numpy as jnp
from jax import lax
from jax.experimental import pallas as pl
from jax.experimental.pallas import tpu as pltpu
from jax.experimental.pallas import tpu_sc as plsc

P, D, DMLP, E = 2048, 768, 3072, 8
TM = 1536
NT = DMLP // TM
NW = 32
PPW = P // NW
LANES = 16
NCHUNK = PPW // LANES


def _gate_body(x_ref, wg_ref, gt_ref):
    gt_ref[...] = lax.dot_general(
        wg_ref[...], x_ref[...], (((0,), (1,)), ((), ())),
        preferred_element_type=jnp.float32)


def _router_body(gt_hbm, out_hbm, gbuf, accv, sem):
    wid = lax.axis_index("s") * 2 + lax.axis_index("c")
    base = wid * PPW
    hs = [pltpu.async_copy(gt_hbm.at[i, pl.ds(base, PPW)], gbuf.at[i], sem)
          for i in range(E)]
    for h in hs:
        h.wait()
    acc = [jnp.zeros((LANES,), jnp.float32) for _ in range(E)]
    for c in range(NCHUNK):
        sl = pl.ds(c * LANES, LANES)
        vs = [gbuf[i, sl] for i in range(E)]
        m1 = vs[0]
        i1 = jnp.zeros((LANES,), jnp.int32)
        for i in range(1, E):
            gt = vs[i] > m1
            i1 = jnp.where(gt, i, i1)
            m1 = jnp.where(gt, vs[i], m1)
        m2 = jnp.full((LANES,), -jnp.inf, jnp.float32)
        i2 = jnp.zeros((LANES,), jnp.int32)
        for i in range(E):
            gt = (vs[i] > m2) & (i1 != i)
            i2 = jnp.where(gt, i, i2)
            m2 = jnp.where(gt, vs[i], m2)
        r = jnp.exp(m2 - m1)
        w1 = 1.0 / (1.0 + r)
        w2 = r / (1.0 + r)
        for i in range(E):
            acc[i] = (acc[i] + jnp.where(i1 == i, w1, 0.0)
                      + jnp.where(i2 == i, w2, 0.0))
    for i in range(E):
        accv[i, pl.ds(0, LANES)] = acc[i]
    pltpu.sync_copy(accv, out_hbm.at[wid])


def _moe_body(x_ref, coef_ref, win_ref, bin_ref, wout_ref, bout_ref,
              out_ref, cbuf_ref):
    e = pl.program_id(0)
    t = pl.program_id(1)

    @pl.when((e == 0) & (t == 0))
    def _init():
        row = jnp.sum(coef_ref[...], axis=0, keepdims=True)
        grp = jax.lax.broadcasted_iota(jnp.int32, (1, E * LANES), 1) // LANES
        cs = []
        for i in range(E):
            ci = jnp.sum(jnp.where(grp == i, row, 0.0),
                         axis=1, keepdims=True)
            cs.append(ci)
            cbuf_ref[i:i + 1, :] = jnp.broadcast_to(ci, (1, 128))
        bias = cs[0] * bout_ref[0:1, :]
        for i in range(1, E):
            bias = bias + cs[i] * bout_ref[i:i + 1, :]
        out_ref[...] = jnp.broadcast_to(bias, out_ref.shape)

    c11 = cbuf_ref[pl.ds(e, 1), 0:1]
    pre = jnp.dot(x_ref[...], win_ref[0],
                  preferred_element_type=jnp.float32) + bin_ref[0]
    h = jnp.maximum(pre, 0.0)
    out_ref[...] += jnp.dot(h, wout_ref[0] * c11,
                            preferred_element_type=jnp.float32)


@jax.jit
def kernel(x, W_gate, W_in, b_in, W_out, b_out):
    B = x.shape[0]
    x2 = x.reshape(B * P, D)
    b_in3 = b_in.reshape(E, 1, DMLP)

    gt = pl.pallas_call(
        _gate_body,
        in_specs=[pl.BlockSpec((B * P, D), lambda: (0, 0)),
                  pl.BlockSpec((D, E), lambda: (0, 0))],
        out_specs=pl.BlockSpec((E, B * P), lambda: (0, 0)),
        out_shape=jax.ShapeDtypeStruct((E, B * P), jnp.float32),
        grid=(),
    )(x2, W_gate)

    router = pl.kernel(
        _router_body,
        mesh=plsc.VectorSubcoreMesh(core_axis_name="c", subcore_axis_name="s"),
        out_type=jax.ShapeDtypeStruct((NW, E, LANES), jnp.float32),
        scratch_types=[
            pltpu.VMEM((E, PPW), jnp.float32),
            pltpu.VMEM((E, LANES), jnp.float32),
            pltpu.SemaphoreType.DMA,
        ],
    )
    partial_coefs = router(gt).reshape(NW, E * LANES)

    out = pl.pallas_call(
        _moe_body,
        grid=(E, NT),
        in_specs=[
            pl.BlockSpec((B * P, D), lambda e, t: (0, 0)),
            pl.BlockSpec((NW, E * LANES), lambda e, t: (0, 0)),
            pl.BlockSpec((1, D, TM), lambda e, t: (e, 0, t)),
            pl.BlockSpec((1, 1, TM), lambda e, t: (e, 0, t)),
            pl.BlockSpec((1, TM, D), lambda e, t: (e, t, 0)),
            pl.BlockSpec((E, D), lambda e, t: (0, 0)),
        ],
        out_specs=pl.BlockSpec((B * P, D), lambda e, t: (0, 0)),
        out_shape=jax.ShapeDtypeStruct((B * P, D), jnp.float32),
        scratch_shapes=[pltpu.VMEM((E, 128), jnp.float32)],
        compiler_params=pltpu.CompilerParams(
            dimension_semantics=("arbitrary", "arbitrary")),
    )(x2, partial_coefs, W_in, b_in3, W_out, b_out)
    return out.reshape(B, P, D)
